# baseline (speedup 1.0000x reference)
import functools

import jax
import jax.numpy as jnp
from jax import lax
from jax.experimental import pallas as pl
from jax.experimental.pallas import tpu as pltpu
from jax.experimental.pallas import tpu_sc as plsc

_N = 10000
_E = 320000
_D = 128
_NC = 2
_NS = 16
_CH = 128
_EP = 327680
_CPT = (_EP // _CH) // _NS
_RC = 80
_NRC = _N // _RC
_NP = 10240
_NPA = 10080
_DC = 128


def _sc_mesh():
  return plsc.VectorSubcoreMesh(core_axis_name="c", subcore_axis_name="s")


def _write_out(acc, out_hbm, cid, sid):
  for k in range(8):
    c = sid + 16 * k

    @pl.when(c < _NRC)
    def _():
      pltpu.sync_copy(acc.at[pl.ds(c * _RC, _RC)],
                      out_hbm.at[cid, pl.ds(c * _RC, _RC)])


def _sc_degree(dst2d, ones128, zeros128):
  @functools.partial(
      pl.kernel,
      out_type=jax.ShapeDtypeStruct((2 * _NP,), jnp.float32),
      mesh=_sc_mesh(),
      scratch_types=[
          pltpu.VMEM_SHARED((_NP,), jnp.float32),
          pltpu.VMEM((_CH,), jnp.float32),
          pltpu.VMEM((_CH,), jnp.float32),
          pltpu.VMEM((_CPT, _CH), jnp.int32),
          pltpu.SemaphoreType.DMA,
      ],
  )
  def k(dst_hbm, ones_hbm, zeros_hbm, out_hbm, dacc, obuf, zbuf, di, sem):
    cid = lax.axis_index("c")
    sid = lax.axis_index("s")
    pltpu.sync_copy(ones_hbm, obuf)
    pltpu.sync_copy(zeros_hbm, zbuf)
    for kk in range(_NP // _DC // _NS):
      pltpu.sync_copy(zbuf, dacc.at[pl.ds((sid + 16 * kk) * _DC, _DC)])
    row0 = cid * (_EP // _CH) + sid * _CPT
    pltpu.sync_copy(dst_hbm.at[pl.ds(row0, _CPT)], di)
    plsc.subcore_barrier()

    @pl.loop(0, _CPT)
    def _(j):
      pltpu.async_copy(obuf, dacc.at[di.at[j]], sem, add=True)

    @pl.loop(0, _CPT)
    def _(j):
      pltpu.make_async_copy(obuf, dacc.at[di.at[0]], sem).wait()

    plsc.subcore_barrier()
    for kk in range(_NP // _DC // _NS):
      c = (sid + 16 * kk) * _DC
      pltpu.sync_copy(dacc.at[pl.ds(c, _DC)],
                      out_hbm.at[pl.ds(cid * _NP + c, _DC)])

  return k(dst2d, ones128, zeros128)


def _sc_scatter_rows(q_cat, src_f, dst_f, zeros128):
  @functools.partial(
      pl.kernel,
      out_type=jax.ShapeDtypeStruct((2, _N, _D), jnp.float32),
      mesh=_sc_mesh(),
      scratch_types=[
          pltpu.VMEM_SHARED((_NPA, _D), jnp.float32),
          [pltpu.VMEM((_CH, _D), jnp.float32)] * 3,
          [pltpu.VMEM((_CH,), jnp.int32)] * 3,
          [pltpu.VMEM((_CH,), jnp.int32)] * 3,
          [pltpu.SemaphoreType.DMA] * 3,
          [pltpu.SemaphoreType.DMA] * 3,
      ],
  )
  def k(q_hbm, src_hbm, dst_hbm, z_hbm, out_hbm, acc,
        rows, si, di, sem_g, sem_s):
    cid = lax.axis_index("c")
    sid = lax.axis_index("s")
    pltpu.sync_copy(z_hbm, rows[0])
    for kk in range(8):
      c = sid + 16 * kk

      @pl.when(c < _NPA // _RC)
      def _():
        pltpu.sync_copy(rows[0].at[pl.ds(0, _RC)],
                        acc.at[pl.ds(c * _RC, _RC)])

    e0 = cid * _EP + sid * _CPT * _CH
    plsc.subcore_barrier()

    def load_idx(j, p):
      pltpu.sync_copy(src_hbm.at[pl.ds(e0 + j * _CH, _CH)], si[p])
      pltpu.sync_copy(dst_hbm.at[pl.ds(e0 + j * _CH, _CH)], di[p])

    for p in range(2):
      load_idx(p, p)
      pltpu.async_copy(q_hbm.at[si[p]], rows[p], sem_g[p])

    @pl.loop(0, _CPT - 1, step=3)
    def _(i):
      for b in range(3):
        j = i + b
        pltpu.make_async_copy(q_hbm.at[si[b]], rows[b], sem_g[b]).wait()
        pltpu.async_copy(rows[b], acc.at[di[b]], sem_s[b], add=True)
        p2 = (b + 2) % 3
        j2 = j + 2

        @pl.when(j2 < _CPT)
        def _():
          @pl.when(j2 >= 3)
          def _():
            pltpu.make_async_copy(rows[p2], acc.at[di[p2]], sem_s[p2]).wait()

          load_idx(j2, p2)
          pltpu.async_copy(q_hbm.at[si[p2]], rows[p2], sem_g[p2])

    pltpu.make_async_copy(q_hbm.at[si[0]], rows[0], sem_g[0]).wait()
    pltpu.async_copy(rows[0], acc.at[di[0]], sem_s[0], add=True)
    for b in range(3):
      pltpu.make_async_copy(rows[b], acc.at[di[b]], sem_s[b]).wait()

    plsc.subcore_barrier()
    _write_out(acc, out_hbm, cid, sid)

  return k(q_cat, src_f, dst_f, zeros128)


_R = 2000
_NT = _N // _R


def _tc_embed_scale(x, w_emb, b_emb, w_g0, deg3):
  def body(x_ref, we_ref, be_ref, wg_ref, dg_ref, o_ref, r_ref):
    i = pl.program_id(1)
    h = jnp.dot(x_ref[0], we_ref[0], preferred_element_type=jnp.float32)
    h = h + be_ref[0]
    dg = dg_ref[0, pl.ds(i, 1), :][0]
    r = lax.rsqrt(jnp.maximum(dg, 1.0))
    rc = jnp.reshape(r, (_R, 1))
    rb = jnp.broadcast_to(rc, (_R, _D))
    r_ref[0] = rb
    o_ref[0] = jnp.dot(h * rb, wg_ref[0], preferred_element_type=jnp.float32)

  return pl.pallas_call(
      body,
      grid=(2, _NT),
      in_specs=[
          pl.BlockSpec((1, _R, _D), lambda b, i: (b, i, 0)),
          pl.BlockSpec((1, _D, _D), lambda b, i: (b, 0, 0)),
          pl.BlockSpec((1, 1, _D), lambda b, i: (b, 0, 0)),
          pl.BlockSpec((1, _D, _D), lambda b, i: (b, 0, 0)),
          pl.BlockSpec((1, _NT, _R), lambda b, i: (b, 0, 0)),
      ],
      out_specs=[
          pl.BlockSpec((1, _R, _D), lambda b, i: (b, i, 0)),
          pl.BlockSpec((1, _R, _D), lambda b, i: (b, i, 0)),
      ],
      out_shape=[
          jax.ShapeDtypeStruct((2, _N, _D), jnp.float32),
          jax.ShapeDtypeStruct((2, _N, _D), jnp.float32),
      ],
  )(x, w_emb, b_emb, w_g0, deg3)


def _tc_layer_mid(agg, rbig, b_prev, w_next):
  def body(a_ref, r_ref, bp_ref, wn_ref, o_ref):
    r = r_ref[0]
    h = jnp.maximum(a_ref[0] * r + bp_ref[0], 0.0)
    o_ref[0] = jnp.dot(h * r, wn_ref[0], preferred_element_type=jnp.float32)

  return pl.pallas_call(
      body,
      grid=(2, _NT),
      in_specs=[
          pl.BlockSpec((1, _R, _D), lambda b, i: (b, i, 0)),
          pl.BlockSpec((1, _R, _D), lambda b, i: (b, i, 0)),
          pl.BlockSpec((1, 1, _D), lambda b, i: (b, 0, 0)),
          pl.BlockSpec((1, _D, _D), lambda b, i: (b, 0, 0)),
      ],
      out_specs=pl.BlockSpec((1, _R, _D), lambda b, i: (b, i, 0)),
      out_shape=jax.ShapeDtypeStruct((2, _N, _D), jnp.float32),
  )(agg, rbig, b_prev, w_next)


def _tc_layer_max(agg, rbig, b_prev):
  def body(a_ref, r_ref, bp_ref, o_ref):
    i = pl.program_id(1)
    h = jnp.maximum(a_ref[0] * r_ref[0] + bp_ref[0], 0.0)
    pm = jnp.max(h, axis=0)[None, None, :]

    @pl.when(i == 0)
    def _():
      o_ref[...] = jnp.full((1, 1, _D), -jnp.inf, jnp.float32)

    o_ref[...] = jnp.maximum(o_ref[...], pm)

  return pl.pallas_call(
      body,
      grid=(2, _NT),
      in_specs=[
          pl.BlockSpec((1, _R, _D), lambda b, i: (b, i, 0)),
          pl.BlockSpec((1, _R, _D), lambda b, i: (b, i, 0)),
          pl.BlockSpec((1, 1, _D), lambda b, i: (b, 0, 0)),
      ],
      out_specs=pl.BlockSpec((1, 1, _D), lambda b, i: (b, 0, 0)),
      out_shape=jax.ShapeDtypeStruct((2, 1, _D), jnp.float32),
  )(agg, rbig, b_prev)


def _tc_head(m, w_f_pad, b_f_pad):
  def body(m_ref, w_ref, b_ref, o_ref):
    feats = jnp.concatenate([m_ref[0], m_ref[1]], axis=-1)
    logits = jnp.dot(feats, w_ref[...], preferred_element_type=jnp.float32)
    logits = logits + b_ref[...]
    act = jnp.where(logits >= 0.0, logits, 0.01 * logits)
    lane = lax.broadcasted_iota(jnp.int32, (1, _D), 1)
    masked = jnp.where(lane < 2, act, -jnp.inf)
    mx = jnp.max(masked, axis=-1, keepdims=True)
    e = jnp.where(lane < 2, jnp.exp(masked - mx), 0.0)
    o_ref[...] = e / jnp.sum(e, axis=-1, keepdims=True)

  return pl.pallas_call(
      body,
      out_shape=jax.ShapeDtypeStruct((1, _D), jnp.float32),
  )(m, w_f_pad, b_f_pad)


def kernel(x1, edge_index1, x2, edge_index2, W_emb1, b_emb1, W_emb2, b_emb2,
           W_g1_0, b_g1_0, W_g1_1, b_g1_1, W_g2_0, b_g2_0, W_g2_1, b_g2_1,
           W_f, b_f):
  src1 = edge_index1[0].astype(jnp.int32)
  dst1 = edge_index1[1].astype(jnp.int32)
  src2 = edge_index2[0].astype(jnp.int32)
  dst2 = edge_index2[1].astype(jnp.int32)
  pad_s = jnp.zeros((_EP - _E,), jnp.int32)
  pad_d = jnp.full((_EP - _E,), _NPA - 1, jnp.int32)
  src_f = jnp.concatenate([src1, pad_s, src2 + _N, pad_s])
  dst_f = jnp.concatenate([dst1, pad_d, dst2, pad_d])
  dst2d = dst_f.reshape(-1, _CH)

  ones1 = jnp.ones((_CH,), jnp.float32)
  zeros1 = jnp.zeros((_CH,), jnp.float32)
  zeros128 = jnp.zeros((_CH, _D), jnp.float32)

  x = jnp.stack([x1, x2])
  w_emb = jnp.stack([W_emb1, W_emb2])
  b_emb = jnp.stack([b_emb1, b_emb2])[:, None, :]
  w_l0 = jnp.stack([W_g1_0, W_g2_0])
  b_l0 = jnp.stack([b_g1_0, b_g2_0])[:, None, :]
  w_l1 = jnp.stack([W_g1_1, W_g2_1])
  b_l1 = jnp.stack([b_g1_1, b_g2_1])[:, None, :]

  degf = _sc_degree(dst2d, ones1, zeros1)
  deg3 = degf.reshape(2, _NP)[:, :_N].reshape(2, _NT, _R)

  q0, rbig = _tc_embed_scale(x, w_emb, b_emb, w_l0, deg3)
  agg0 = _sc_scatter_rows(q0.reshape(2 * _N, _D), src_f, dst_f, zeros128)
  q1 = _tc_layer_mid(agg0, rbig, b_l0, w_l1)
  agg1 = _sc_scatter_rows(q1.reshape(2 * _N, _D), src_f, dst_f, zeros128)
  m = _tc_layer_max(agg1, rbig, b_l1)

  w_f_pad = jnp.zeros((2 * _D, _D), jnp.float32).at[:, :2].set(W_f)
  b_f_pad = jnp.zeros((1, _D), jnp.float32).at[:, :2].set(b_f[None, :])
  out = _tc_head(m, w_f_pad, b_f_pad)
  return out[:, :2]
